# baseline (speedup 1.0000x reference)
import jax
import jax.numpy as jnp
from jax import lax
from jax.experimental import pallas as pl
from jax.experimental.pallas import tpu as pltpu
from jax.experimental.pallas import tpu_sc as plsc

H = 128
B = 1048576
NC = 2
NS = 16
NW = NC * NS
L = 16

PER_W = B // NW
CHUNK = 8192
N_CHUNKS = PER_W // CHUNK


def _sc_body(x_hbm, inds_hbm, rad_hbm, out_hbm, x_v, flat_v, idx_v, rows_v,
             sem_idx, sem_rows):
    wid = lax.axis_index("s") * NC + lax.axis_index("c")
    lanes = lax.iota(jnp.int32, L)
    fH = jnp.float32(H)
    zero = jnp.int32(0)
    hi = jnp.int32(H - 1)

    for c in range(N_CHUNKS):
        pbase = wid * PER_W + c * CHUNK
        pltpu.sync_copy(x_hbm.at[pl.ds(3 * pbase, 3 * CHUNK)], x_v)

        def compute(g, _):
            off = pl.multiple_of(g * L, L)
            base3 = off * 3 + lanes * 3
            gx = plsc.load_gather(x_v, [base3])
            gy = plsc.load_gather(x_v, [base3 + 1])
            gz = plsc.load_gather(x_v, [base3 + 2])
            xi = jnp.clip((gx * fH).astype(jnp.int32), zero, hi)
            yi = jnp.clip((gy * fH).astype(jnp.int32), zero, hi)
            zi = jnp.clip((gz * fH).astype(jnp.int32), zero, hi)
            flat_v[pl.ds(off, L)] = (zi * H + yi) * H + xi
            return 0

        lax.fori_loop(0, CHUNK // L, compute, 0, unroll=4)

        pltpu.async_copy(inds_hbm.at[flat_v], idx_v, sem_idx).wait()
        pltpu.async_copy(rad_hbm.at[idx_v], rows_v, sem_rows).wait()
        pltpu.sync_copy(rows_v, out_hbm.at[pl.ds(pbase, CHUNK), :])


@jax.jit
def _voxel_slf(x_flat, inds_p1, rad_pad):
    mesh = plsc.VectorSubcoreMesh(core_axis_name="c", subcore_axis_name="s")
    run = pl.kernel(
        _sc_body,
        out_type=jax.ShapeDtypeStruct((B, 3), jnp.float32),
        mesh=mesh,
        scratch_types=[
            pltpu.VMEM((3 * CHUNK,), jnp.float32),
            pltpu.VMEM((CHUNK,), jnp.int32),
            pltpu.VMEM((CHUNK,), jnp.int32),
            pltpu.VMEM((CHUNK, 3), jnp.float32),
            pltpu.SemaphoreType.DMA,
            pltpu.SemaphoreType.DMA,
        ],
        compiler_params=pltpu.CompilerParams(
            needs_layout_passes=False, use_tc_tiling_on_sc=False),
    )
    return run(x_flat, inds_p1, rad_pad)


def kernel(x, inds, radiance):
    x_flat = x.reshape(-1)
    inds_p1 = inds.reshape(-1).astype(jnp.int32) + 1
    rad_pad = jnp.concatenate(
        [jnp.zeros((1, 3), jnp.float32), radiance], axis=0)
    return _voxel_slf(x_flat, inds_p1, rad_pad)
